# baseline (speedup 1.0000x reference)
import functools

import jax
import jax.numpy as jnp
from jax import lax
from jax.experimental import pallas as pl
from jax.experimental.pallas import tpu as pltpu
from jax.experimental.pallas import tpu_sc as plsc

EMB = 32
BATCH = 16384
CHUNK = 128
LANES = 16


@functools.cache
def _build():
    info = plsc.get_sparse_core_info()
    nw = info.num_cores * info.num_subcores
    n = BATCH // nw
    nch = n // CHUNK

    mesh = plsc.VectorSubcoreMesh(core_axis_name="c", subcore_axis_name="s")

    @functools.partial(
        pl.kernel,
        mesh=mesh,
        out_type=jax.ShapeDtypeStruct((2 * BATCH, EMB), jnp.float32),
        compiler_params=pltpu.CompilerParams(
            use_tc_tiling_on_sc=False, skip_device_barrier=True
        ),
        scratch_types=[
            pltpu.VMEM((nch, CHUNK), jnp.int32),
            pltpu.VMEM((nch, CHUNK), jnp.int32),
            pltpu.VMEM((nch, CHUNK), jnp.int32),
            pltpu.VMEM((nch, CHUNK), jnp.int32),
            pltpu.VMEM((n, EMB), jnp.float32),
            pltpu.VMEM((n, EMB), jnp.float32),
            pltpu.SemaphoreType.DMA,
        ],
    )
    def k(idx_a_hbm, idx_b_hbm, wa_hbm, wb_hbm, out_hbm,
          ia_v, ib_v, oa_v, ob_v, ra_v, rb_v, sem):
        wid = lax.axis_index("s") * info.num_cores + lax.axis_index("c")
        base = wid * n
        cp_a = pltpu.async_copy(idx_a_hbm.at[pl.ds(wid * nch, nch)], ia_v, sem)
        cp_b = pltpu.async_copy(idx_b_hbm.at[pl.ds(wid * nch, nch)], ib_v, sem)
        iot2 = lax.iota(jnp.int32, LANES) * 2
        for j in range(nch):
            for v in range(CHUNK // LANES):
                s = 2 * (base + j * CHUNK + v * LANES)
                oa_v[j, pl.ds(v * LANES, LANES)] = iot2 + s
                ob_v[j, pl.ds(v * LANES, LANES)] = iot2 + (s + 1)
        cp_a.wait()
        cp_b.wait()
        gathers = []
        for j in range(nch):
            gathers.append(
                pltpu.async_copy(
                    wa_hbm.at[ia_v.at[j]],
                    ra_v.at[pl.ds(j * CHUNK, CHUNK)], sem)
            )
            gathers.append(
                pltpu.async_copy(
                    wb_hbm.at[ib_v.at[j]],
                    rb_v.at[pl.ds(j * CHUNK, CHUNK)], sem)
            )
        for c in gathers:
            c.wait()
        scatters = []
        for j in range(nch):
            scatters.append(
                pltpu.async_copy(ra_v.at[pl.ds(j * CHUNK, CHUNK)], out_hbm.at[oa_v.at[j]], sem)
            )
            scatters.append(
                pltpu.async_copy(rb_v.at[pl.ds(j * CHUNK, CHUNK)], out_hbm.at[ob_v.at[j]], sem)
            )
        for c in scatters:
            c.wait()

    return k


def kernel(field_a, field_b, W_a, W_b):
    k = _build()
    ia = field_a.reshape(BATCH // CHUNK, CHUNK)
    ib = field_b.reshape(BATCH // CHUNK, CHUNK)
    out2 = k(ia, ib, W_a, W_b)
    return out2.reshape(BATCH, 2 * EMB)


# speedup vs baseline: 1.0011x; 1.0011x over previous
import functools

import jax
import jax.numpy as jnp
from jax import lax
from jax.experimental import pallas as pl
from jax.experimental.pallas import tpu as pltpu
from jax.experimental.pallas import tpu_sc as plsc

EMB = 32
BATCH = 16384
CHUNK = 128
LANES = 16


@functools.cache
def _build():
    info = plsc.get_sparse_core_info()
    nw = info.num_cores * info.num_subcores
    n = BATCH // nw
    nch = n // CHUNK

    mesh = plsc.VectorSubcoreMesh(core_axis_name="c", subcore_axis_name="s")

    @functools.partial(
        pl.kernel,
        mesh=mesh,
        out_type=jax.ShapeDtypeStruct((2 * BATCH, EMB), jnp.float32),
        compiler_params=pltpu.CompilerParams(use_tc_tiling_on_sc=False),
        scratch_types=[
            pltpu.VMEM((nch, CHUNK), jnp.int32),
            pltpu.VMEM((nch, CHUNK), jnp.int32),
            pltpu.VMEM((nch, CHUNK), jnp.int32),
            pltpu.VMEM((nch, CHUNK), jnp.int32),
            pltpu.VMEM((n, EMB), jnp.float32),
            pltpu.VMEM((n, EMB), jnp.float32),
            pltpu.SemaphoreType.DMA,
        ],
    )
    def k(idx_a_hbm, idx_b_hbm, wa_hbm, wb_hbm, out_hbm,
          ia_v, ib_v, oa_v, ob_v, ra_v, rb_v, sem):
        wid = lax.axis_index("s") * info.num_cores + lax.axis_index("c")
        base = wid * n
        cp_a = pltpu.async_copy(idx_a_hbm.at[pl.ds(wid * nch, nch)], ia_v, sem)
        cp_b = pltpu.async_copy(idx_b_hbm.at[pl.ds(wid * nch, nch)], ib_v, sem)
        iot2 = lax.iota(jnp.int32, LANES) * 2
        for j in range(nch):
            for v in range(CHUNK // LANES):
                s = 2 * (base + j * CHUNK + v * LANES)
                oa_v[j, pl.ds(v * LANES, LANES)] = iot2 + s
                ob_v[j, pl.ds(v * LANES, LANES)] = iot2 + (s + 1)
        cp_a.wait()
        cp_b.wait()
        gathers = []
        for j in range(nch):
            gathers.append(
                pltpu.async_copy(
                    wa_hbm.at[ia_v.at[j]],
                    ra_v.at[pl.ds(j * CHUNK, CHUNK)], sem)
            )
            gathers.append(
                pltpu.async_copy(
                    wb_hbm.at[ib_v.at[j]],
                    rb_v.at[pl.ds(j * CHUNK, CHUNK)], sem)
            )
        for c in gathers:
            c.wait()
        scatters = []
        for j in range(nch):
            scatters.append(
                pltpu.async_copy(ra_v.at[pl.ds(j * CHUNK, CHUNK)], out_hbm.at[oa_v.at[j]], sem)
            )
            scatters.append(
                pltpu.async_copy(rb_v.at[pl.ds(j * CHUNK, CHUNK)], out_hbm.at[ob_v.at[j]], sem)
            )
        for c in scatters:
            c.wait()

    return k


def kernel(field_a, field_b, W_a, W_b):
    k = _build()
    ia = field_a.reshape(BATCH // CHUNK, CHUNK)
    ib = field_b.reshape(BATCH // CHUNK, CHUNK)
    out2 = k(ia, ib, W_a, W_b)
    return out2.reshape(BATCH, 2 * EMB)
